# Initial kernel scaffold; baseline (speedup 1.0000x reference)
import jax
import jax.numpy as jnp
from jax.experimental import pallas as pl


def kernel(x, W_learner, b_learner, W_sel1, b_sel1, W_sel2, b_sel2, W_ens, b_ens):
    raise NotImplementedError("write your pallas kernel here")


# speedup vs baseline: 1.4755x; 1.4755x over previous
import jax
import jax.numpy as jnp
from jax.experimental import pallas as pl
from jax.experimental.pallas import tpu as pltpu

N = 8192
D = 1024
E = 8
H = 16
BN = 256


def _fused_body(x_ref, wl_ref, bl_ref, ws1_ref, bs1_ref, ws2_ref, bs2_ref,
                wens_ref, bens_ref, out_ref):
    x = x_ref[...]
    lo = jnp.dot(x, wl_ref[...], preferred_element_type=jnp.float32)
    lo = lo + bl_ref[...]
    h = jnp.maximum(jnp.dot(lo, ws1_ref[...],
                            preferred_element_type=jnp.float32)
                    + bs1_ref[...], 0.0)
    logits = jnp.dot(h, ws2_ref[...],
                     preferred_element_type=jnp.float32) + bs2_ref[...]
    m = jnp.max(logits, axis=-1, keepdims=True)
    ids = jax.lax.broadcasted_iota(jnp.int32, logits.shape, 1)
    first = jnp.min(jnp.where(logits == m, ids, E), axis=-1, keepdims=True)
    gmask = jnp.where(ids == first, logits, 0.0)

    acc = lo
    for e in range(E):
        g = gmask[:, e:e + 1]
        acc = acc + jnp.dot(x * g, wens_ref[e],
                            preferred_element_type=jnp.float32)
        acc = acc + g * bens_ref[e]
    out_ref[...] = acc


def kernel(x, W_learner, b_learner, W_sel1, b_sel1, W_sel2, b_sel2, W_ens, b_ens):
    grid = (N // BN,)
    resident = lambda *shape: pl.BlockSpec(shape, lambda i: (0,) * len(shape))
    out = pl.pallas_call(
        _fused_body,
        grid=grid,
        in_specs=[
            pl.BlockSpec((BN, D), lambda i: (i, 0)),
            resident(D, D),
            resident(D),
            resident(D, H),
            resident(H),
            resident(H, E),
            resident(E),
            resident(E, D, D),
            resident(E, D),
        ],
        out_specs=pl.BlockSpec((BN, D), lambda i: (i, 0)),
        out_shape=jax.ShapeDtypeStruct((N, D), jnp.float32),
    )(x, W_learner, b_learner, W_sel1, b_sel1, W_sel2, b_sel2, W_ens, b_ens)
    return out


# speedup vs baseline: 1.4943x; 1.0128x over previous
import jax
import jax.numpy as jnp
from jax.experimental import pallas as pl
from jax.experimental.pallas import tpu as pltpu

N = 8192
D = 1024
E = 8
H = 16
BN = 256


def _fused_body(x_ref, wl_ref, wlbf_ref, bl_ref, ws1_ref, bs1_ref, ws2_ref,
                bs2_ref, wensbf_ref, bens_ref, out_ref, wf_ref, bf_ref):
    i = pl.program_id(0)

    @pl.when(i == 0)
    def _():
        wf_ref[...] = jnp.dot(wl_ref[...], ws1_ref[...],
                              preferred_element_type=jnp.float32)
        bf_ref[...] = (jnp.dot(bl_ref[...].reshape(1, D), ws1_ref[...],
                               preferred_element_type=jnp.float32)
                       + bs1_ref[...].reshape(1, H))

    x = x_ref[...]
    h = jnp.maximum(jnp.dot(x, wf_ref[...],
                            preferred_element_type=jnp.float32)
                    + bf_ref[...], 0.0)
    logits = jnp.dot(h, ws2_ref[...],
                     preferred_element_type=jnp.float32) + bs2_ref[...]
    m = jnp.max(logits, axis=-1, keepdims=True)
    ids = jax.lax.broadcasted_iota(jnp.int32, logits.shape, 1)
    first = jnp.min(jnp.where(logits == m, ids, E), axis=-1, keepdims=True)
    gmask = jnp.where(ids == first, logits, 0.0)

    xbf = x.astype(jnp.bfloat16)
    acc = jnp.dot(xbf, wlbf_ref[...],
                  preferred_element_type=jnp.float32) + bl_ref[...]
    for e in range(E):
        g = gmask[:, e:e + 1]
        xg = (x * g).astype(jnp.bfloat16)
        acc = acc + jnp.dot(xg, wensbf_ref[e],
                            preferred_element_type=jnp.float32)
        acc = acc + g * bens_ref[e]
    out_ref[...] = acc


def kernel(x, W_learner, b_learner, W_sel1, b_sel1, W_sel2, b_sel2, W_ens, b_ens):
    grid = (N // BN,)
    resident = lambda *shape: pl.BlockSpec(shape, lambda i: (0,) * len(shape))
    out = pl.pallas_call(
        _fused_body,
        grid=grid,
        in_specs=[
            pl.BlockSpec((BN, D), lambda i: (i, 0)),
            resident(D, D),
            resident(D, D),
            resident(D),
            resident(D, H),
            resident(H),
            resident(H, E),
            resident(E),
            resident(E, D, D),
            resident(E, D),
        ],
        out_specs=pl.BlockSpec((BN, D), lambda i: (i, 0)),
        out_shape=jax.ShapeDtypeStruct((N, D), jnp.float32),
        scratch_shapes=[
            pltpu.VMEM((D, H), jnp.float32),
            pltpu.VMEM((1, H), jnp.float32),
        ],
    )(x, W_learner, W_learner.astype(jnp.bfloat16), b_learner, W_sel1, b_sel1,
      W_sel2, b_sel2, W_ens.astype(jnp.bfloat16), b_ens)
    return out
